# Initial kernel scaffold; baseline (speedup 1.0000x reference)
import jax
import jax.numpy as jnp
from jax.experimental import pallas as pl


def kernel(x, edge_index, edge_weight, mlp_kernel0, mlp_bias0, neigh_kernel0, self_kernel0, bias0, mlp_kernel1, mlp_bias1, neigh_kernel1, self_kernel1, bias1, gcn_kernel, gcn_bias):
    raise NotImplementedError("write your pallas kernel here")


# speedup vs baseline: 1.7181x; 1.7181x over previous
import functools

import jax
import jax.numpy as jnp
from jax.experimental import pallas as pl
from jax.experimental.pallas import tpu as pltpu

N = 10000
NPAD = 10240


def _dense_block(x_ref, *refs):
    pass


def _l1_transform_body(x_ref, mk0_ref, mb0_ref, sk0_ref, xm0_ref, fx0_ref):
    x = x_ref[...]
    xm0_ref[...] = jnp.maximum(
        jnp.dot(x, mk0_ref[...], preferred_element_type=jnp.float32) + mb0_ref[...], 0.0
    )
    fx0_ref[...] = jnp.dot(x, sk0_ref[...], preferred_element_type=jnp.float32)


def _l1_transform(x, mk0, mb0, sk0):
    BN = 1024
    grid = (NPAD // BN,)
    return pl.pallas_call(
        _l1_transform_body,
        grid=grid,
        in_specs=[
            pl.BlockSpec((BN, 128), lambda i: (i, 0)),
            pl.BlockSpec((128, 64), lambda i: (0, 0)),
            pl.BlockSpec((1, 64), lambda i: (0, 0)),
            pl.BlockSpec((128, 64), lambda i: (0, 0)),
        ],
        out_specs=[
            pl.BlockSpec((BN, 64), lambda i: (i, 0)),
            pl.BlockSpec((BN, 64), lambda i: (i, 0)),
        ],
        out_shape=[
            jax.ShapeDtypeStruct((NPAD, 64), jnp.float32),
            jax.ShapeDtypeStruct((NPAD, 64), jnp.float32),
        ],
    )(x, mk0, mb0.reshape(1, 64), sk0)


def _l1_finish_body(fx0_ref, red_ref, nk0_ref, b0_ref, mk1_ref, mb1_ref,
                    sk1_ref, h1_ref, xm1_ref, fx1_ref):
    red = jnp.maximum(red_ref[...], 0.0)
    fn = jnp.dot(red, nk0_ref[...], preferred_element_type=jnp.float32)
    h1 = jnp.maximum(
        jnp.concatenate([fx0_ref[...], fn], axis=1) + b0_ref[...], 0.0
    )
    h1_ref[...] = h1
    xm1_ref[...] = jnp.maximum(
        jnp.dot(h1, mk1_ref[...], preferred_element_type=jnp.float32) + mb1_ref[...], 0.0
    )
    fx1_ref[...] = jnp.dot(h1, sk1_ref[...], preferred_element_type=jnp.float32)


def _l1_finish(fx0, red0, nk0, b0, mk1, mb1, sk1):
    BN = 1024
    grid = (NPAD // BN,)
    return pl.pallas_call(
        _l1_finish_body,
        grid=grid,
        in_specs=[
            pl.BlockSpec((BN, 64), lambda i: (i, 0)),
            pl.BlockSpec((BN, 64), lambda i: (i, 0)),
            pl.BlockSpec((64, 64), lambda i: (0, 0)),
            pl.BlockSpec((1, 128), lambda i: (0, 0)),
            pl.BlockSpec((128, 32), lambda i: (0, 0)),
            pl.BlockSpec((1, 32), lambda i: (0, 0)),
            pl.BlockSpec((128, 32), lambda i: (0, 0)),
        ],
        out_specs=[
            pl.BlockSpec((BN, 128), lambda i: (i, 0)),
            pl.BlockSpec((BN, 32), lambda i: (i, 0)),
            pl.BlockSpec((BN, 32), lambda i: (i, 0)),
        ],
        out_shape=[
            jax.ShapeDtypeStruct((NPAD, 128), jnp.float32),
            jax.ShapeDtypeStruct((NPAD, 32), jnp.float32),
            jax.ShapeDtypeStruct((NPAD, 32), jnp.float32),
        ],
    )(fx0, red0, nk0, b0.reshape(1, 128), mk1, mb1.reshape(1, 32), sk1)


def _l2_finish_body(fx1_ref, red_ref, nk1_ref, b1_ref, gk_ref, deg_ref,
                    h3s_ref, dis_ref):
    red = jnp.maximum(red_ref[...], 0.0)
    fn = jnp.dot(red, nk1_ref[...], preferred_element_type=jnp.float32)
    h2 = jnp.maximum(
        jnp.concatenate([fx1_ref[...], fn], axis=1) + b1_ref[...], 0.0
    )
    h3 = jnp.dot(h2, gk_ref[...], preferred_element_type=jnp.float32)
    deg = deg_ref[...]
    dis = jnp.where(deg > 0.0, jax.lax.rsqrt(jnp.maximum(deg, 1e-30)), 0.0)
    h3s_ref[...] = h3 * dis
    dis_ref[...] = dis


def _l2_finish(fx1, red1, nk1, b1, gk_pad, deg):
    BN = 1024
    grid = (NPAD // BN,)
    return pl.pallas_call(
        _l2_finish_body,
        grid=grid,
        in_specs=[
            pl.BlockSpec((BN, 32), lambda i: (i, 0)),
            pl.BlockSpec((BN, 32), lambda i: (i, 0)),
            pl.BlockSpec((32, 32), lambda i: (0, 0)),
            pl.BlockSpec((1, 64), lambda i: (0, 0)),
            pl.BlockSpec((64, 128), lambda i: (0, 0)),
            pl.BlockSpec((BN, 1), lambda i: (i, 0)),
        ],
        out_specs=[
            pl.BlockSpec((BN, 128), lambda i: (i, 0)),
            pl.BlockSpec((BN, 1), lambda i: (i, 0)),
        ],
        out_shape=[
            jax.ShapeDtypeStruct((NPAD, 128), jnp.float32),
            jax.ShapeDtypeStruct((NPAD, 1), jnp.float32),
        ],
    )(fx1, red1, nk1, b1.reshape(1, 64), gk_pad, deg.reshape(NPAD, 1))


def _gcn_finish_body(acc_ref, h3s_ref, dis_ref, gb_ref, out_ref):
    out_ref[...] = acc_ref[...] + dis_ref[...] * h3s_ref[...] + gb_ref[...]


def _gcn_finish(acc, h3s, dis, gb_pad):
    BN = 1024
    grid = (NPAD // BN,)
    return pl.pallas_call(
        _gcn_finish_body,
        grid=grid,
        in_specs=[
            pl.BlockSpec((BN, 128), lambda i: (i, 0)),
            pl.BlockSpec((BN, 128), lambda i: (i, 0)),
            pl.BlockSpec((BN, 1), lambda i: (i, 0)),
            pl.BlockSpec((1, 128), lambda i: (0, 0)),
        ],
        out_specs=pl.BlockSpec((BN, 128), lambda i: (i, 0)),
        out_shape=jax.ShapeDtypeStruct((NPAD, 128), jnp.float32),
    )(acc, h3s, dis.reshape(NPAD, 1), gb_pad)


def kernel(x, edge_index, edge_weight,
           mlp_kernel0, mlp_bias0, neigh_kernel0, self_kernel0, bias0,
           mlp_kernel1, mlp_bias1, neigh_kernel1, self_kernel1, bias1,
           gcn_kernel, gcn_bias):
    row, col = edge_index[0], edge_index[1]
    xpad = jnp.zeros((NPAD, 128), jnp.float32).at[:N].set(x)

    xm0, fx0 = _l1_transform(xpad, mlp_kernel0, mlp_bias0, self_kernel0)
    red0 = jax.ops.segment_max(xm0[col], row, num_segments=NPAD)
    red0 = jnp.maximum(red0, 0.0)

    h1, xm1, fx1 = _l1_finish(fx0, red0, neigh_kernel0, bias0,
                              mlp_kernel1, mlp_bias1, self_kernel1)
    red1 = jax.ops.segment_max(xm1[col], row, num_segments=NPAD)
    red1 = jnp.maximum(red1, 0.0)

    deg = jax.ops.segment_sum(edge_weight, row, num_segments=NPAD)
    deg = deg.at[:N].add(1.0)

    gk_pad = jnp.zeros((64, 128), jnp.float32).at[:, :40].set(gcn_kernel)
    h3s, dis = _l2_finish(fx1, red1, neigh_kernel1, bias1, gk_pad, deg)
    dis = dis[:, 0]

    we = edge_weight * dis[col]
    acc = jax.ops.segment_sum(h3s[row] * we[:, None], col, num_segments=NPAD)

    gb_pad = jnp.zeros((1, 128), jnp.float32).at[0, :40].set(gcn_bias)
    out = _gcn_finish(acc, h3s, dis, gb_pad)
    return out[:N, :40]


# speedup vs baseline: 7.3649x; 4.2867x over previous
import functools

import jax
import jax.numpy as jnp
from jax import lax
from jax.experimental import pallas as pl
from jax.experimental.pallas import tpu as pltpu
from jax.experimental.pallas import tpu_sc as plsc

N = 10000
NPAD = 10240
NGATH = 16384
E = 320000
NQ = 4
EQ = E // NQ
NH = 8
HN = NPAD // NH
B = 400
LCAP = EQ
NBMAX = LCAP // B
NCHUNK = 32
ECH = E // NCHUNK
GCB = ECH // B

_mesh = functools.partial(
    plsc.VectorSubcoreMesh, core_axis_name="c", subcore_axis_name="s",
    num_cores=2, num_subcores=16)

_SC_PARAMS = pltpu.CompilerParams(
    needs_layout_passes=False, use_tc_tiling_on_sc=False)


def _l1_transform_body(x_ref, mk0_ref, mb0_ref, sk0_ref, xm0_ref, fx0_ref):
    x = x_ref[...]
    xm0_ref[...] = jnp.maximum(
        jnp.dot(x, mk0_ref[...], preferred_element_type=jnp.float32)
        + mb0_ref[...], 0.0)
    fx0_ref[...] = jnp.dot(x, sk0_ref[...], preferred_element_type=jnp.float32)


def _l1_transform(x, mk0, mb0, sk0):
    BN = 1024
    return pl.pallas_call(
        _l1_transform_body,
        grid=(NPAD // BN,),
        in_specs=[
            pl.BlockSpec((BN, 128), lambda i: (i, 0)),
            pl.BlockSpec((128, 64), lambda i: (0, 0)),
            pl.BlockSpec((1, 64), lambda i: (0, 0)),
            pl.BlockSpec((128, 64), lambda i: (0, 0)),
        ],
        out_specs=[
            pl.BlockSpec((BN, 64), lambda i: (i, 0)),
            pl.BlockSpec((BN, 64), lambda i: (i, 0)),
        ],
        out_shape=[
            jax.ShapeDtypeStruct((NGATH, 64), jnp.float32),
            jax.ShapeDtypeStruct((NPAD, 64), jnp.float32),
        ],
    )(x, mk0, mb0.reshape(1, 64), sk0)


def _l1_finish_body(fx0_ref, part_ref, nk0_ref, b0_ref, mk1_ref, mb1_ref,
                    sk1_ref, xm1_ref, fx1_ref):
    red = jnp.max(part_ref[...], axis=0)
    fn = jnp.dot(red, nk0_ref[...], preferred_element_type=jnp.float32)
    h1 = jnp.maximum(
        jnp.concatenate([fx0_ref[...], fn], axis=1) + b0_ref[...], 0.0)
    xm1_ref[...] = jnp.maximum(
        jnp.dot(h1, mk1_ref[...], preferred_element_type=jnp.float32)
        + mb1_ref[...], 0.0)
    fx1_ref[...] = jnp.dot(h1, sk1_ref[...], preferred_element_type=jnp.float32)


def _l1_finish(fx0, part0, nk0, b0, mk1, mb1, sk1):
    BN = 1024
    return pl.pallas_call(
        _l1_finish_body,
        grid=(NPAD // BN,),
        in_specs=[
            pl.BlockSpec((BN, 64), lambda i: (i, 0)),
            pl.BlockSpec((NQ, BN, 64), lambda i: (0, i, 0)),
            pl.BlockSpec((64, 64), lambda i: (0, 0)),
            pl.BlockSpec((1, 128), lambda i: (0, 0)),
            pl.BlockSpec((128, 32), lambda i: (0, 0)),
            pl.BlockSpec((1, 32), lambda i: (0, 0)),
            pl.BlockSpec((128, 32), lambda i: (0, 0)),
        ],
        out_specs=[
            pl.BlockSpec((BN, 32), lambda i: (i, 0)),
            pl.BlockSpec((BN, 32), lambda i: (i, 0)),
        ],
        out_shape=[
            jax.ShapeDtypeStruct((NGATH, 32), jnp.float32),
            jax.ShapeDtypeStruct((NPAD, 32), jnp.float32),
        ],
    )(fx0, part0, nk0, b0.reshape(1, 128), mk1, mb1.reshape(1, 32), sk1)


def _l2_finish_body(fx1_ref, part_ref, degp_ref, nk1_ref, b1_ref, gk_ref,
                    h3s_ref, dis_ref):
    i = pl.program_id(0)
    red = jnp.max(part_ref[...], axis=0)
    fn = jnp.dot(red, nk1_ref[...], preferred_element_type=jnp.float32)
    h2 = jnp.maximum(
        jnp.concatenate([fx1_ref[...], fn], axis=1) + b1_ref[...], 0.0)
    h3 = jnp.dot(h2, gk_ref[...], preferred_element_type=jnp.float32)
    bn = fx1_ref.shape[0]
    rows = i * bn + jax.lax.broadcasted_iota(jnp.int32, (bn, 1), 0)
    deg = jnp.sum(degp_ref[...], axis=0) + jnp.where(rows < N, 1.0, 0.0)
    dis = jnp.where(deg > 0.0, jax.lax.rsqrt(jnp.maximum(deg, 1e-30)), 0.0)
    h3s_ref[...] = h3 * dis
    dis_ref[...] = dis


def _l2_finish(fx1, part1, degp, nk1, b1, gk_pad):
    BN = 1024
    return pl.pallas_call(
        _l2_finish_body,
        grid=(NPAD // BN,),
        in_specs=[
            pl.BlockSpec((BN, 32), lambda i: (i, 0)),
            pl.BlockSpec((NQ, BN, 32), lambda i: (0, i, 0)),
            pl.BlockSpec((NQ, BN, 1), lambda i: (0, i, 0)),
            pl.BlockSpec((32, 32), lambda i: (0, 0)),
            pl.BlockSpec((1, 64), lambda i: (0, 0)),
            pl.BlockSpec((64, 48), lambda i: (0, 0)),
        ],
        out_specs=[
            pl.BlockSpec((BN, 48), lambda i: (i, 0)),
            pl.BlockSpec((BN, 1), lambda i: (i, 0)),
        ],
        out_shape=[
            jax.ShapeDtypeStruct((NPAD, 48), jnp.float32),
            jax.ShapeDtypeStruct((NPAD, 1), jnp.float32),
        ],
    )(fx1, part1, degp.reshape(NQ, NPAD, 1), nk1, b1.reshape(1, 64), gk_pad)


def _gcn_finish_body(gp_ref, h3s_ref, dis_ref, gb_ref, out_ref):
    acc = jnp.sum(gp_ref[...], axis=0)
    out_ref[...] = dis_ref[...] * (acc + h3s_ref[...]) + gb_ref[...]


def _gcn_finish(gpart, h3s, dis, gb_pad):
    BN = 1024
    return pl.pallas_call(
        _gcn_finish_body,
        grid=(NPAD // BN,),
        in_specs=[
            pl.BlockSpec((2, BN, 48), lambda i: (0, i, 0)),
            pl.BlockSpec((BN, 48), lambda i: (i, 0)),
            pl.BlockSpec((BN, 1), lambda i: (i, 0)),
            pl.BlockSpec((1, 48), lambda i: (0, 0)),
        ],
        out_specs=pl.BlockSpec((BN, 48), lambda i: (i, 0)),
        out_shape=jax.ShapeDtypeStruct((NPAD, 48), jnp.float32),
    )(gpart, h3s, dis, gb_pad)


def _prepass_body(row_hbm, col_hbm, w_hbm, plist_hbm, cnt_hbm, degp_hbm,
                  rowb, colb, wb, packed, degloc, cntv, sem):
    w = lax.axis_index("c") * 16 + lax.axis_index("s")
    o = w // NH
    h = w % NH
    lo = h * HN
    ebase = o * EQ

    def zero_deg(i, _):
        degloc[pl.ds(i * 16, 16)] = jnp.zeros((16,), jnp.float32)
        return 0
    lax.fori_loop(0, HN // 16, zero_deg, 0)

    def batch(b, n):
        base = ebase + b * B
        pltpu.sync_copy(row_hbm.at[pl.ds(base, B)], rowb)
        pltpu.sync_copy(col_hbm.at[pl.ds(base, B)], colb)
        pltpu.sync_copy(w_hbm.at[pl.ds(base, B)], wb)

        def group(g, n):
            r = rowb[pl.ds(g * 16, 16)]
            c = colb[pl.ds(g * 16, 16)]
            wv = wb[pl.ds(g * 16, 16)]
            rl = r - lo
            m = (r >= lo) & (r < lo + HN)
            pk = c | (rl << 14)
            cum = plsc.cumsum(m.astype(jnp.int32))
            plsc.store_scatter(packed, [n + cum - 1], pk, mask=m)
            plsc.addupdate_scatter(degloc, [rl], wv, mask=m)
            return n + jnp.sum(m.astype(jnp.int32))
        return lax.fori_loop(0, B // 16, group, n)

    n = lax.fori_loop(0, EQ // B, batch, jnp.int32(0))

    pltpu.sync_copy(packed, plist_hbm.at[pl.ds(w * LCAP, LCAP)])
    cntv[...] = jnp.full((16,), n, jnp.int32)
    pltpu.sync_copy(cntv, cnt_hbm.at[pl.ds(w * 16, 16)])
    pltpu.sync_copy(degloc, degp_hbm.at[pl.ds(o * NPAD + lo, HN)])


def _prepass(row, col, ew):
    return pl.kernel(
        _prepass_body,
        out_type=[
            jax.ShapeDtypeStruct((32 * LCAP,), jnp.int32),
            jax.ShapeDtypeStruct((32 * 16,), jnp.int32),
            jax.ShapeDtypeStruct((NQ * NPAD,), jnp.float32),
        ],
        mesh=_mesh(),
        compiler_params=_SC_PARAMS,
        scratch_types=[
            pltpu.VMEM((B,), jnp.int32),
            pltpu.VMEM((B,), jnp.int32),
            pltpu.VMEM((B,), jnp.float32),
            pltpu.VMEM((LCAP,), jnp.int32),
            pltpu.VMEM((HN,), jnp.float32),
            pltpu.VMEM((16,), jnp.int32),
            pltpu.SemaphoreType.DMA,
        ],
    )(row, col, ew)


def _segmax_body(nf, plist_hbm, cnt_hbm, part_hbm, cnt_v, pk_v, col_v,
                 gbuf, acc, sem):
    f = nf.shape[1]
    w = lax.axis_index("c") * 16 + lax.axis_index("s")
    o = w // NH
    h = w % NH

    pltpu.sync_copy(cnt_hbm, cnt_v)
    cnt = cnt_v[pl.ds(w * 16, 16)][0]

    def zero(i, _):
        acc[pl.ds(i * 16, 16)] = jnp.zeros((16,), jnp.float32)
        return 0
    lax.fori_loop(0, HN * f // 16, zero, 0)

    def batch(b, _):
        @pl.when(b * B < cnt)
        def _():
            pltpu.sync_copy(plist_hbm.at[pl.ds(w * LCAP + b * B, B)],
                            pk_v.at[pl.ds(0, B)])

            def unpk(g, _):
                col_v[pl.ds(g * 16, 16)] = pk_v[pl.ds(g * 16, 16)] & 16383
                return 0
            lax.fori_loop(0, B // 16, unpk, 0)
            pltpu.async_copy(nf.at[col_v], gbuf, sem).wait()
            m = jnp.minimum(B, cnt - b * B)

            def edge(j, _):
                rl = pk_v[pl.ds(j, 16)][0] >> 14
                a = rl * f
                for k in range(f // 16):
                    acc[pl.ds(a + k * 16, 16)] = jnp.maximum(
                        acc[pl.ds(a + k * 16, 16)],
                        gbuf[j, pl.ds(k * 16, 16)])
                return 0
            lax.fori_loop(0, m, edge, 0)
        return 0

    lax.fori_loop(0, NBMAX, batch, 0)
    pltpu.sync_copy(acc, part_hbm.at[pl.ds((o * NPAD + h * HN) * f, HN * f)])


def _segmax(nf, plist, cnt, f):
    return pl.kernel(
        _segmax_body,
        out_type=jax.ShapeDtypeStruct((NQ * NPAD * f,), jnp.float32),
        mesh=_mesh(),
        compiler_params=_SC_PARAMS,
        scratch_types=[
            pltpu.VMEM((32 * 16,), jnp.int32),
            pltpu.VMEM((B + 16,), jnp.int32),
            pltpu.VMEM((B,), jnp.int32),
            pltpu.VMEM((B, f), jnp.float32),
            pltpu.VMEM((HN * f,), jnp.float32),
            pltpu.SemaphoreType.DMA,
        ],
    )(nf, plist, cnt)


def _gcn_body(h3s_hbm, row_hbm, col_hbm, w_hbm, gpart_hbm,
              row_v, col_v, w_v, gbuf, zb, acc_sh, sem):
    c = lax.axis_index("c")
    s = lax.axis_index("s")
    w = c * 16 + s
    ebase = w * ECH

    def zzb(i, _):
        for k in range(3):
            zb[i, pl.ds(k * 16, 16)] = jnp.zeros((16,), jnp.float32)
        return 0
    lax.fori_loop(0, 320, zzb, 0)
    rows0 = s * (NPAD // 16)
    pltpu.sync_copy(zb, acc_sh.at[pl.ds(rows0, 320)])
    pltpu.sync_copy(zb, acc_sh.at[pl.ds(rows0 + 320, 320)])
    plsc.subcore_barrier()

    def batch(b, _):
        base = ebase + b * B
        pltpu.sync_copy(row_hbm.at[pl.ds(base, B)], row_v)
        pltpu.sync_copy(col_hbm.at[pl.ds(base, B)], col_v)
        pltpu.sync_copy(w_hbm.at[pl.ds(base, B)], w_v.at[pl.ds(0, B)])
        pltpu.async_copy(h3s_hbm.at[row_v], gbuf, sem).wait()

        def edge(j, _):
            wj = w_v[pl.ds(j, 16)][0]
            for k in range(3):
                gbuf[j, pl.ds(k * 16, 16)] = gbuf[j, pl.ds(k * 16, 16)] * wj
            return 0
        lax.fori_loop(0, B, edge, 0)
        pltpu.sync_copy(gbuf, acc_sh.at[col_v], add=True)
        return 0

    lax.fori_loop(0, GCB, batch, 0)
    plsc.subcore_barrier()
    pltpu.sync_copy(acc_sh.at[pl.ds(rows0, NPAD // 16)],
                    gpart_hbm.at[c, pl.ds(rows0, NPAD // 16)])


def _gcn_edges(h3s, row, col, ew):
    return pl.kernel(
        _gcn_body,
        out_type=jax.ShapeDtypeStruct((2, NPAD, 48), jnp.float32),
        mesh=_mesh(),
        compiler_params=_SC_PARAMS,
        scratch_types=[
            pltpu.VMEM((B,), jnp.int32),
            pltpu.VMEM((B,), jnp.int32),
            pltpu.VMEM((B + 16,), jnp.float32),
            pltpu.VMEM((B, 48), jnp.float32),
            pltpu.VMEM((320, 48), jnp.float32),
            pltpu.VMEM_SHARED((NPAD, 48), jnp.float32),
            pltpu.SemaphoreType.DMA,
        ],
    )(h3s, row, col, ew)


def kernel(x, edge_index, edge_weight,
           mlp_kernel0, mlp_bias0, neigh_kernel0, self_kernel0, bias0,
           mlp_kernel1, mlp_bias1, neigh_kernel1, self_kernel1, bias1,
           gcn_kernel, gcn_bias):
    row, col = edge_index[0], edge_index[1]
    xpad = jnp.zeros((NPAD, 128), jnp.float32).at[:N].set(x)

    plist, cnt, degp = _prepass(row, col, edge_weight)

    xm0, fx0 = _l1_transform(xpad, mlp_kernel0, mlp_bias0, self_kernel0)
    part0 = _segmax(xm0, plist, cnt, 64).reshape(NQ, NPAD, 64)

    xm1, fx1 = _l1_finish(fx0, part0, neigh_kernel0, bias0,
                          mlp_kernel1, mlp_bias1, self_kernel1)
    part1 = _segmax(xm1, plist, cnt, 32).reshape(NQ, NPAD, 32)

    gk_pad = jnp.zeros((64, 48), jnp.float32).at[:, :40].set(gcn_kernel)
    h3s, dis = _l2_finish(fx1, part1, degp.reshape(NQ, NPAD), neigh_kernel1,
                          bias1, gk_pad)

    gpart = _gcn_edges(h3s, row, col, edge_weight)

    gb_pad = jnp.zeros((1, 48), jnp.float32).at[0, :40].set(gcn_bias)
    out = _gcn_finish(gpart, h3s, dis, gb_pad)
    return out[:N, :40]


# speedup vs baseline: 9.5951x; 1.3028x over previous
import functools

import jax
import jax.numpy as jnp
from jax import lax
from jax.experimental import pallas as pl
from jax.experimental.pallas import tpu as pltpu
from jax.experimental.pallas import tpu_sc as plsc

N = 10000
NPAD = 10240
NGATH = 16384
E = 320000
NQ = 4
EQ = E // NQ
NH = 8
HN = NPAD // NH
PB = 4000
LCAP = EQ
NCHUNK = 32
ECH = E // NCHUNK
GB = 1000
GCB = ECH // GB

_mesh = functools.partial(
    plsc.VectorSubcoreMesh, core_axis_name="c", subcore_axis_name="s",
    num_cores=2, num_subcores=16)

_SC_PARAMS = pltpu.CompilerParams(
    needs_layout_passes=False, use_tc_tiling_on_sc=False)


def _l1_transform_body(x_ref, mk0_ref, mb0_ref, sk0_ref, xm0_ref, fx0_ref):
    x = x_ref[...]
    xm0_ref[...] = jnp.maximum(
        jnp.dot(x, mk0_ref[...], preferred_element_type=jnp.float32)
        + mb0_ref[...], 0.0)
    fx0_ref[...] = jnp.dot(x, sk0_ref[...], preferred_element_type=jnp.float32)


def _l1_transform(x, mk0, mb0, sk0):
    BN = 1024
    return pl.pallas_call(
        _l1_transform_body,
        grid=(NPAD // BN,),
        in_specs=[
            pl.BlockSpec((BN, 128), lambda i: (i, 0)),
            pl.BlockSpec((128, 64), lambda i: (0, 0)),
            pl.BlockSpec((1, 64), lambda i: (0, 0)),
            pl.BlockSpec((128, 64), lambda i: (0, 0)),
        ],
        out_specs=[
            pl.BlockSpec((BN, 64), lambda i: (i, 0)),
            pl.BlockSpec((BN, 64), lambda i: (i, 0)),
        ],
        out_shape=[
            jax.ShapeDtypeStruct((NGATH, 64), jnp.float32),
            jax.ShapeDtypeStruct((NPAD, 64), jnp.float32),
        ],
    )(x, mk0, mb0.reshape(1, 64), sk0)


def _l1_finish_body(fx0_ref, part_ref, nk0_ref, b0_ref, mk1_ref, mb1_ref,
                    sk1_ref, xm1_ref, fx1_ref):
    red = jnp.max(part_ref[...], axis=0)
    fn = jnp.dot(red, nk0_ref[...], preferred_element_type=jnp.float32)
    h1 = jnp.maximum(
        jnp.concatenate([fx0_ref[...], fn], axis=1) + b0_ref[...], 0.0)
    xm1_ref[...] = jnp.maximum(
        jnp.dot(h1, mk1_ref[...], preferred_element_type=jnp.float32)
        + mb1_ref[...], 0.0)
    fx1_ref[...] = jnp.dot(h1, sk1_ref[...], preferred_element_type=jnp.float32)


def _l1_finish(fx0, part0, nk0, b0, mk1, mb1, sk1):
    BN = 1024
    return pl.pallas_call(
        _l1_finish_body,
        grid=(NPAD // BN,),
        in_specs=[
            pl.BlockSpec((BN, 64), lambda i: (i, 0)),
            pl.BlockSpec((NQ, BN, 64), lambda i: (0, i, 0)),
            pl.BlockSpec((64, 64), lambda i: (0, 0)),
            pl.BlockSpec((1, 128), lambda i: (0, 0)),
            pl.BlockSpec((128, 32), lambda i: (0, 0)),
            pl.BlockSpec((1, 32), lambda i: (0, 0)),
            pl.BlockSpec((128, 32), lambda i: (0, 0)),
        ],
        out_specs=[
            pl.BlockSpec((BN, 32), lambda i: (i, 0)),
            pl.BlockSpec((BN, 32), lambda i: (i, 0)),
        ],
        out_shape=[
            jax.ShapeDtypeStruct((NGATH, 32), jnp.float32),
            jax.ShapeDtypeStruct((NPAD, 32), jnp.float32),
        ],
    )(fx0, part0, nk0, b0.reshape(1, 128), mk1, mb1.reshape(1, 32), sk1)


def _l2_finish_body(fx1_ref, part_ref, degp_ref, nk1_ref, b1_ref, gk_ref,
                    h3s_ref, dis_ref):
    i = pl.program_id(0)
    red = jnp.max(part_ref[...], axis=0)
    fn = jnp.dot(red, nk1_ref[...], preferred_element_type=jnp.float32)
    h2 = jnp.maximum(
        jnp.concatenate([fx1_ref[...], fn], axis=1) + b1_ref[...], 0.0)
    h3 = jnp.dot(h2, gk_ref[...], preferred_element_type=jnp.float32)
    bn = fx1_ref.shape[0]
    rows = i * bn + jax.lax.broadcasted_iota(jnp.int32, (bn, 1), 0)
    deg = jnp.sum(degp_ref[...], axis=0) + jnp.where(rows < N, 1.0, 0.0)
    dis = jnp.where(deg > 0.0, jax.lax.rsqrt(jnp.maximum(deg, 1e-30)), 0.0)
    h3s_ref[...] = h3 * dis
    dis_ref[...] = dis


def _l2_finish(fx1, part1, degp, nk1, b1, gk_pad):
    BN = 1024
    return pl.pallas_call(
        _l2_finish_body,
        grid=(NPAD // BN,),
        in_specs=[
            pl.BlockSpec((BN, 32), lambda i: (i, 0)),
            pl.BlockSpec((NQ, BN, 32), lambda i: (0, i, 0)),
            pl.BlockSpec((NQ, BN, 1), lambda i: (0, i, 0)),
            pl.BlockSpec((32, 32), lambda i: (0, 0)),
            pl.BlockSpec((1, 64), lambda i: (0, 0)),
            pl.BlockSpec((64, 48), lambda i: (0, 0)),
        ],
        out_specs=[
            pl.BlockSpec((BN, 48), lambda i: (i, 0)),
            pl.BlockSpec((BN, 1), lambda i: (i, 0)),
        ],
        out_shape=[
            jax.ShapeDtypeStruct((NPAD, 48), jnp.float32),
            jax.ShapeDtypeStruct((NPAD, 1), jnp.float32),
        ],
    )(fx1, part1, degp.reshape(NQ, NPAD, 1), nk1, b1.reshape(1, 64), gk_pad)


def _gcn_finish_body(gp_ref, h3s_ref, dis_ref, gb_ref, out_ref):
    acc = jnp.sum(gp_ref[...], axis=0)
    out_ref[...] = dis_ref[...] * (acc + h3s_ref[...]) + gb_ref[...]


def _gcn_finish(gpart, h3s, dis, gb_pad):
    BN = 1024
    return pl.pallas_call(
        _gcn_finish_body,
        grid=(NPAD // BN,),
        in_specs=[
            pl.BlockSpec((2, BN, 48), lambda i: (0, i, 0)),
            pl.BlockSpec((BN, 48), lambda i: (i, 0)),
            pl.BlockSpec((BN, 1), lambda i: (i, 0)),
            pl.BlockSpec((1, 48), lambda i: (0, 0)),
        ],
        out_specs=pl.BlockSpec((BN, 48), lambda i: (i, 0)),
        out_shape=jax.ShapeDtypeStruct((NPAD, 48), jnp.float32),
    )(gpart, h3s, dis, gb_pad)


def _prepass_body(row_hbm, col_hbm, w_hbm, plist_hbm, cnt_hbm, degp_hbm,
                  rowb, colb, wb, packed, degloc, cntv, sem):
    w = lax.axis_index("c") * 16 + lax.axis_index("s")
    o = w // NH
    h = w % NH
    lo = h * HN
    ebase = o * EQ

    def zero_deg(i, _):
        degloc[pl.ds(i * 16, 16)] = jnp.zeros((16,), jnp.float32)
        return 0
    lax.fori_loop(0, HN // 16, zero_deg, 0)

    def batch(b, n):
        base = ebase + b * PB
        d1 = pltpu.async_copy(row_hbm.at[pl.ds(base, PB)], rowb, sem)
        d2 = pltpu.async_copy(col_hbm.at[pl.ds(base, PB)], colb, sem)
        d3 = pltpu.async_copy(w_hbm.at[pl.ds(base, PB)], wb, sem)
        d1.wait()
        d2.wait()
        d3.wait()

        def group(g, n):
            r = rowb[pl.ds(g * 16, 16)]
            c = colb[pl.ds(g * 16, 16)]
            wv = wb[pl.ds(g * 16, 16)]
            rl = r - lo
            m = (r >= lo) & (r < lo + HN)
            pk = c | (rl << 14)
            cum = plsc.cumsum(m.astype(jnp.int32))
            plsc.store_scatter(packed, [n + cum - 1], pk, mask=m)
            plsc.addupdate_scatter(degloc, [rl], wv, mask=m)
            return n + plsc.all_reduce_population_count(m)[0]
        return lax.fori_loop(0, PB // 16, group, n)

    n = lax.fori_loop(0, EQ // PB, batch, jnp.int32(0))

    pltpu.sync_copy(packed, plist_hbm.at[pl.ds(w * LCAP, LCAP)])
    cntv[...] = jnp.full((16,), n, jnp.int32)
    pltpu.sync_copy(cntv, cnt_hbm.at[pl.ds(w * 16, 16)])
    pltpu.sync_copy(degloc, degp_hbm.at[pl.ds(o * NPAD + lo, HN)])


def _prepass(row, col, ew):
    return pl.kernel(
        _prepass_body,
        out_type=[
            jax.ShapeDtypeStruct((32 * LCAP + 1024,), jnp.int32),
            jax.ShapeDtypeStruct((32 * 16,), jnp.int32),
            jax.ShapeDtypeStruct((NQ * NPAD,), jnp.float32),
        ],
        mesh=_mesh(),
        compiler_params=_SC_PARAMS,
        scratch_types=[
            pltpu.VMEM((PB,), jnp.int32),
            pltpu.VMEM((PB,), jnp.int32),
            pltpu.VMEM((PB,), jnp.float32),
            pltpu.VMEM((LCAP,), jnp.int32),
            pltpu.VMEM((HN,), jnp.float32),
            pltpu.VMEM((16,), jnp.int32),
            pltpu.SemaphoreType.DMA,
        ],
    )(row, col, ew)


def _segmax_body(nf, plist_hbm, cnt_hbm, part_hbm, cnt_v, pk_v, col_v,
                 gbuf, acc, sem):
    f = nf.shape[1]
    bsz = gbuf.shape[0]
    nb = -(-LCAP // bsz)
    w = lax.axis_index("c") * 16 + lax.axis_index("s")
    o = w // NH
    h = w % NH

    pltpu.sync_copy(cnt_hbm, cnt_v)
    cnt = cnt_v[pl.ds(w * 16, 16)][0]

    def zero(i, _):
        acc[pl.ds(i * 16, 16)] = jnp.zeros((16,), jnp.float32)
        return 0
    lax.fori_loop(0, HN * f // 16, zero, 0)

    def batch(b, _):
        @pl.when(b * bsz < cnt)
        def _():
            pltpu.sync_copy(plist_hbm.at[pl.ds(w * LCAP + b * bsz, bsz)],
                            pk_v.at[pl.ds(0, bsz)])

            def unpk(g, _):
                col_v[pl.ds(g * 16, 16)] = pk_v[pl.ds(g * 16, 16)] & 16383
                return 0
            lax.fori_loop(0, bsz // 16, unpk, 0)
            pltpu.async_copy(nf.at[col_v], gbuf, sem).wait()
            m = jnp.minimum(bsz, cnt - b * bsz)

            def edge(j, _):
                rl = pk_v[pl.ds(j, 16)][0] >> 14
                a = rl * f
                for k in range(f // 16):
                    acc[pl.ds(a + k * 16, 16)] = jnp.maximum(
                        acc[pl.ds(a + k * 16, 16)],
                        gbuf[j, pl.ds(k * 16, 16)])
                return 0
            lax.fori_loop(0, m, edge, 0)
        return 0

    lax.fori_loop(0, nb, batch, 0)
    pltpu.sync_copy(acc, part_hbm.at[pl.ds((o * NPAD + h * HN) * f, HN * f)])


def _segmax(nf, plist, cnt, f):
    bsz = 512 if f == 64 else 1024
    return pl.kernel(
        _segmax_body,
        out_type=jax.ShapeDtypeStruct((NQ * NPAD * f,), jnp.float32),
        mesh=_mesh(),
        compiler_params=_SC_PARAMS,
        scratch_types=[
            pltpu.VMEM((32 * 16,), jnp.int32),
            pltpu.VMEM((bsz + 16,), jnp.int32),
            pltpu.VMEM((bsz,), jnp.int32),
            pltpu.VMEM((bsz, f), jnp.float32),
            pltpu.VMEM((HN * f,), jnp.float32),
            pltpu.SemaphoreType.DMA,
        ],
    )(nf, plist, cnt)


def _gcn_body(h3s_hbm, row_hbm, col_hbm, w_hbm, gpart_hbm,
              row_v, col_v, w_v, gbuf, zb, acc_sh, sem):
    c = lax.axis_index("c")
    s = lax.axis_index("s")
    w = c * 16 + s
    ebase = w * ECH

    def zzb(i, _):
        for k in range(3):
            zb[i, pl.ds(k * 16, 16)] = jnp.zeros((16,), jnp.float32)
        return 0
    lax.fori_loop(0, 320, zzb, 0)
    rows0 = s * (NPAD // 16)
    pltpu.sync_copy(zb, acc_sh.at[pl.ds(rows0, 320)])
    pltpu.sync_copy(zb, acc_sh.at[pl.ds(rows0 + 320, 320)])
    plsc.subcore_barrier()

    def batch(b, _):
        base = ebase + b * GB
        d1 = pltpu.async_copy(row_hbm.at[pl.ds(base, GB)], row_v, sem)
        d2 = pltpu.async_copy(col_hbm.at[pl.ds(base, GB)], col_v, sem)
        d3 = pltpu.async_copy(w_hbm.at[pl.ds(base, GB)], w_v.at[pl.ds(0, GB)],
                              sem)
        d1.wait()
        d2.wait()
        d3.wait()
        pltpu.async_copy(h3s_hbm.at[row_v], gbuf, sem).wait()

        def edge(j, _):
            wj = w_v[pl.ds(j, 16)][0]
            for k in range(3):
                gbuf[j, pl.ds(k * 16, 16)] = gbuf[j, pl.ds(k * 16, 16)] * wj
            return 0
        lax.fori_loop(0, GB, edge, 0)
        pltpu.sync_copy(gbuf, acc_sh.at[col_v], add=True)
        return 0

    lax.fori_loop(0, GCB, batch, 0)
    plsc.subcore_barrier()
    pltpu.sync_copy(acc_sh.at[pl.ds(rows0, NPAD // 16)],
                    gpart_hbm.at[c, pl.ds(rows0, NPAD // 16)])


def _gcn_edges(h3s, row, col, ew):
    return pl.kernel(
        _gcn_body,
        out_type=jax.ShapeDtypeStruct((2, NPAD, 48), jnp.float32),
        mesh=_mesh(),
        compiler_params=_SC_PARAMS,
        scratch_types=[
            pltpu.VMEM((GB,), jnp.int32),
            pltpu.VMEM((GB,), jnp.int32),
            pltpu.VMEM((GB + 16,), jnp.float32),
            pltpu.VMEM((GB, 48), jnp.float32),
            pltpu.VMEM((320, 48), jnp.float32),
            pltpu.VMEM_SHARED((NPAD, 48), jnp.float32),
            pltpu.SemaphoreType.DMA,
        ],
    )(h3s, row, col, ew)


def kernel(x, edge_index, edge_weight,
           mlp_kernel0, mlp_bias0, neigh_kernel0, self_kernel0, bias0,
           mlp_kernel1, mlp_bias1, neigh_kernel1, self_kernel1, bias1,
           gcn_kernel, gcn_bias):
    row, col = edge_index[0], edge_index[1]
    xpad = jnp.zeros((NPAD, 128), jnp.float32).at[:N].set(x)

    plist, cnt, degp = _prepass(row, col, edge_weight)

    xm0, fx0 = _l1_transform(xpad, mlp_kernel0, mlp_bias0, self_kernel0)
    part0 = _segmax(xm0, plist, cnt, 64).reshape(NQ, NPAD, 64)

    xm1, fx1 = _l1_finish(fx0, part0, neigh_kernel0, bias0,
                          mlp_kernel1, mlp_bias1, self_kernel1)
    part1 = _segmax(xm1, plist, cnt, 32).reshape(NQ, NPAD, 32)

    gk_pad = jnp.zeros((64, 48), jnp.float32).at[:, :40].set(gcn_kernel)
    h3s, dis = _l2_finish(fx1, part1, degp.reshape(NQ, NPAD), neigh_kernel1,
                          bias1, gk_pad)

    gpart = _gcn_edges(h3s, row, col, edge_weight)

    gb_pad = jnp.zeros((1, 48), jnp.float32).at[0, :40].set(gcn_bias)
    out = _gcn_finish(gpart, h3s, dis, gb_pad)
    return out[:N, :40]


# speedup vs baseline: 13.7778x; 1.4359x over previous
import functools

import jax
import jax.numpy as jnp
from jax import lax
from jax.experimental import pallas as pl
from jax.experimental.pallas import tpu as pltpu
from jax.experimental.pallas import tpu_sc as plsc

N = 10000
NPAD = 10240
NGATH = 16384
E = 320000
NQ = 4
EQ = E // NQ
NH = 8
HN = NPAD // NH
PB = 4000
LCAP = EQ
NCHUNK = 32
ECH = E // NCHUNK
GB = 1000
GCB = ECH // GB

_mesh = functools.partial(
    plsc.VectorSubcoreMesh, core_axis_name="c", subcore_axis_name="s",
    num_cores=2, num_subcores=16)

_SC_PARAMS = pltpu.CompilerParams(
    needs_layout_passes=False, use_tc_tiling_on_sc=False)


def _l1_transform_body(x_ref, mk0_ref, mb0_ref, sk0_ref, xm0_ref, fx0_ref):
    x = x_ref[...]
    xm0_ref[...] = jnp.maximum(
        jnp.dot(x, mk0_ref[...], preferred_element_type=jnp.float32)
        + mb0_ref[...], 0.0)
    fx0_ref[...] = jnp.dot(x, sk0_ref[...], preferred_element_type=jnp.float32)


def _l1_transform(x, mk0, mb0, sk0):
    BN = 1024
    return pl.pallas_call(
        _l1_transform_body,
        grid=(NPAD // BN,),
        in_specs=[
            pl.BlockSpec((BN, 128), lambda i: (i, 0)),
            pl.BlockSpec((128, 64), lambda i: (0, 0)),
            pl.BlockSpec((1, 64), lambda i: (0, 0)),
            pl.BlockSpec((128, 64), lambda i: (0, 0)),
        ],
        out_specs=[
            pl.BlockSpec((BN, 64), lambda i: (i, 0)),
            pl.BlockSpec((BN, 64), lambda i: (i, 0)),
        ],
        out_shape=[
            jax.ShapeDtypeStruct((NGATH, 64), jnp.float32),
            jax.ShapeDtypeStruct((NPAD, 64), jnp.float32),
        ],
    )(x, mk0, mb0.reshape(1, 64), sk0)


def _l1_finish_body(fx0_ref, part_ref, nk0_ref, b0_ref, mk1_ref, mb1_ref,
                    sk1_ref, xm1_ref, fx1_ref):
    red = jnp.max(part_ref[...], axis=0)
    fn = jnp.dot(red, nk0_ref[...], preferred_element_type=jnp.float32)
    h1 = jnp.maximum(
        jnp.concatenate([fx0_ref[...], fn], axis=1) + b0_ref[...], 0.0)
    xm1_ref[...] = jnp.maximum(
        jnp.dot(h1, mk1_ref[...], preferred_element_type=jnp.float32)
        + mb1_ref[...], 0.0)
    fx1_ref[...] = jnp.dot(h1, sk1_ref[...], preferred_element_type=jnp.float32)


def _l1_finish(fx0, part0, nk0, b0, mk1, mb1, sk1):
    BN = 1024
    return pl.pallas_call(
        _l1_finish_body,
        grid=(NPAD // BN,),
        in_specs=[
            pl.BlockSpec((BN, 64), lambda i: (i, 0)),
            pl.BlockSpec((NQ, BN, 64), lambda i: (0, i, 0)),
            pl.BlockSpec((64, 64), lambda i: (0, 0)),
            pl.BlockSpec((1, 128), lambda i: (0, 0)),
            pl.BlockSpec((128, 32), lambda i: (0, 0)),
            pl.BlockSpec((1, 32), lambda i: (0, 0)),
            pl.BlockSpec((128, 32), lambda i: (0, 0)),
        ],
        out_specs=[
            pl.BlockSpec((BN, 32), lambda i: (i, 0)),
            pl.BlockSpec((BN, 32), lambda i: (i, 0)),
        ],
        out_shape=[
            jax.ShapeDtypeStruct((NGATH, 32), jnp.float32),
            jax.ShapeDtypeStruct((NPAD, 32), jnp.float32),
        ],
    )(fx0, part0, nk0, b0.reshape(1, 128), mk1, mb1.reshape(1, 32), sk1)


def _l2_finish_body(fx1_ref, part_ref, degp_ref, nk1_ref, b1_ref, gk_ref,
                    h3s_ref, dis_ref):
    i = pl.program_id(0)
    red = jnp.max(part_ref[...], axis=0)
    fn = jnp.dot(red, nk1_ref[...], preferred_element_type=jnp.float32)
    h2 = jnp.maximum(
        jnp.concatenate([fx1_ref[...], fn], axis=1) + b1_ref[...], 0.0)
    h3 = jnp.dot(h2, gk_ref[...], preferred_element_type=jnp.float32)
    bn = fx1_ref.shape[0]
    rows = i * bn + jax.lax.broadcasted_iota(jnp.int32, (bn, 1), 0)
    deg = jnp.sum(degp_ref[...], axis=0) + jnp.where(rows < N, 1.0, 0.0)
    dis = jnp.where(deg > 0.0, jax.lax.rsqrt(jnp.maximum(deg, 1e-30)), 0.0)
    h3s_ref[...] = h3 * dis
    dis_ref[...] = dis


def _l2_finish(fx1, part1, degp, nk1, b1, gk_pad):
    BN = 1024
    return pl.pallas_call(
        _l2_finish_body,
        grid=(NPAD // BN,),
        in_specs=[
            pl.BlockSpec((BN, 32), lambda i: (i, 0)),
            pl.BlockSpec((NQ, BN, 32), lambda i: (0, i, 0)),
            pl.BlockSpec((NQ, BN, 1), lambda i: (0, i, 0)),
            pl.BlockSpec((32, 32), lambda i: (0, 0)),
            pl.BlockSpec((1, 64), lambda i: (0, 0)),
            pl.BlockSpec((64, 48), lambda i: (0, 0)),
        ],
        out_specs=[
            pl.BlockSpec((BN, 48), lambda i: (i, 0)),
            pl.BlockSpec((BN, 1), lambda i: (i, 0)),
        ],
        out_shape=[
            jax.ShapeDtypeStruct((NPAD, 48), jnp.float32),
            jax.ShapeDtypeStruct((NPAD, 1), jnp.float32),
        ],
    )(fx1, part1, degp.reshape(NQ, NPAD, 1), nk1, b1.reshape(1, 64), gk_pad)


def _gcn_finish_body(gp_ref, h3s_ref, dis_ref, gb_ref, out_ref):
    acc = jnp.sum(gp_ref[...], axis=0)
    out_ref[...] = dis_ref[...] * (acc + h3s_ref[...]) + gb_ref[...]


def _gcn_finish(gpart, h3s, dis, gb_pad):
    BN = 1024
    return pl.pallas_call(
        _gcn_finish_body,
        grid=(NPAD // BN,),
        in_specs=[
            pl.BlockSpec((2, BN, 48), lambda i: (0, i, 0)),
            pl.BlockSpec((BN, 48), lambda i: (i, 0)),
            pl.BlockSpec((BN, 1), lambda i: (i, 0)),
            pl.BlockSpec((1, 48), lambda i: (0, 0)),
        ],
        out_specs=pl.BlockSpec((BN, 48), lambda i: (i, 0)),
        out_shape=jax.ShapeDtypeStruct((NPAD, 48), jnp.float32),
    )(gpart, h3s, dis, gb_pad)


def _prepass_body(row_hbm, col_hbm, w_hbm, plist_hbm, cnt_hbm, degp_hbm,
                  rowb, colb, wb, packed, degloc, cntv, sem):
    w = lax.axis_index("c") * 16 + lax.axis_index("s")
    o = w // NH
    h = w % NH
    lo = h * HN
    ebase = o * EQ

    def zero_deg(i, _):
        degloc[pl.ds(i * 16, 16)] = jnp.zeros((16,), jnp.float32)
        return 0
    lax.fori_loop(0, HN // 16, zero_deg, 0)

    def batch(b, n):
        base = ebase + b * PB
        d1 = pltpu.async_copy(row_hbm.at[pl.ds(base, PB)], rowb, sem)
        d2 = pltpu.async_copy(col_hbm.at[pl.ds(base, PB)], colb, sem)
        d3 = pltpu.async_copy(w_hbm.at[pl.ds(base, PB)], wb, sem)
        d1.wait()
        d2.wait()
        d3.wait()

        def group(g, n):
            r = rowb[pl.ds(g * 16, 16)]
            c = colb[pl.ds(g * 16, 16)]
            wv = wb[pl.ds(g * 16, 16)]
            rl = r - lo
            m = (r >= lo) & (r < lo + HN)
            pk = c | (rl << 14)
            cum = plsc.cumsum(m.astype(jnp.int32))
            plsc.store_scatter(packed, [n + cum - 1], pk, mask=m)
            plsc.addupdate_scatter(degloc, [rl], wv, mask=m)
            return n + plsc.all_reduce_population_count(m)[0]
        return lax.fori_loop(0, PB // 16, group, n)

    n = lax.fori_loop(0, EQ // PB, batch, jnp.int32(0))

    pltpu.sync_copy(packed, plist_hbm.at[pl.ds(w * LCAP, LCAP)])
    cntv[...] = jnp.full((16,), n, jnp.int32)
    pltpu.sync_copy(cntv, cnt_hbm.at[pl.ds(w * 16, 16)])
    pltpu.sync_copy(degloc, degp_hbm.at[pl.ds(o * NPAD + lo, HN)])


def _prepass(row, col, ew):
    return pl.kernel(
        _prepass_body,
        out_type=[
            jax.ShapeDtypeStruct((32 * LCAP + 1024,), jnp.int32),
            jax.ShapeDtypeStruct((32 * 16,), jnp.int32),
            jax.ShapeDtypeStruct((NQ * NPAD,), jnp.float32),
        ],
        mesh=_mesh(),
        compiler_params=_SC_PARAMS,
        scratch_types=[
            pltpu.VMEM((PB,), jnp.int32),
            pltpu.VMEM((PB,), jnp.int32),
            pltpu.VMEM((PB,), jnp.float32),
            pltpu.VMEM((LCAP,), jnp.int32),
            pltpu.VMEM((HN,), jnp.float32),
            pltpu.VMEM((16,), jnp.int32),
            pltpu.SemaphoreType.DMA,
        ],
    )(row, col, ew)


def _segmax_body(nf, plist_hbm, cnt_hbm, part_hbm, cnt_v, pk_v, col_v,
                 gbuf, acc, sem):
    f = nf.shape[1]
    bsz = gbuf.shape[1]
    nb = -(-LCAP // bsz)
    w = lax.axis_index("c") * 16 + lax.axis_index("s")
    o = w // NH
    h = w % NH

    pltpu.sync_copy(cnt_hbm, cnt_v)
    cnt = cnt_v[pl.ds(w * 16, 16)][0]

    def zero(i, _):
        acc[pl.ds(i * 16, 16)] = jnp.zeros((16,), jnp.float32)
        return 0
    lax.fori_loop(0, HN * f // 16, zero, 0)

    def stage(b, q):
        pltpu.sync_copy(plist_hbm.at[pl.ds(w * LCAP + b * bsz, bsz)],
                        pk_v.at[q, pl.ds(0, bsz)])

        def unpk(g, _):
            col_v[q, pl.ds(g * 16, 16)] = pk_v[q, pl.ds(g * 16, 16)] & 16383
            return 0
        lax.fori_loop(0, bsz // 16, unpk, 0)
        pltpu.async_copy(nf.at[col_v.at[q]], gbuf.at[q], sem)

    @pl.when(cnt > 0)
    def _():
        stage(0, 0)

    def batch(b, _):
        p = b % 2

        @pl.when((b + 1) * bsz < cnt)
        def _():
            stage(b + 1, 1 - p)

        @pl.when(b * bsz < cnt)
        def _():
            pltpu.make_async_copy(nf.at[col_v.at[p]], gbuf.at[p], sem).wait()
            m = jnp.minimum(bsz, cnt - b * bsz)

            def group_acc(g, _):
                pkg = pk_v[p, pl.ds(g * 16, 16)]
                for l in range(16):
                    a = (pkg[l] >> 14) * f
                    j = g * 16 + l
                    for k in range(f // 16):
                        acc[pl.ds(a + k * 16, 16)] = jnp.maximum(
                            acc[pl.ds(a + k * 16, 16)],
                            gbuf[p, j, pl.ds(k * 16, 16)])
                return 0
            lax.fori_loop(0, m // 16, group_acc, 0)

            def edge(j, _):
                rl = pk_v[p, pl.ds(j, 16)][0] >> 14
                a = rl * f
                for k in range(f // 16):
                    acc[pl.ds(a + k * 16, 16)] = jnp.maximum(
                        acc[pl.ds(a + k * 16, 16)],
                        gbuf[p, j, pl.ds(k * 16, 16)])
                return 0
            lax.fori_loop((m // 16) * 16, m, edge, 0)
        return 0

    lax.fori_loop(0, nb, batch, 0)
    pltpu.sync_copy(acc, part_hbm.at[pl.ds((o * NPAD + h * HN) * f, HN * f)])


def _segmax(nf, plist, cnt, f):
    bsz = 256 if f == 64 else 768
    return pl.kernel(
        _segmax_body,
        out_type=jax.ShapeDtypeStruct((NQ * NPAD * f,), jnp.float32),
        mesh=_mesh(),
        compiler_params=_SC_PARAMS,
        scratch_types=[
            pltpu.VMEM((32 * 16,), jnp.int32),
            pltpu.VMEM((2, bsz + 16), jnp.int32),
            pltpu.VMEM((2, bsz), jnp.int32),
            pltpu.VMEM((2, bsz, f), jnp.float32),
            pltpu.VMEM((HN * f,), jnp.float32),
            pltpu.SemaphoreType.DMA,
        ],
    )(nf, plist, cnt)


def _gcn_body(h3s_hbm, row_hbm, col_hbm, w_hbm, gpart_hbm,
              row_v, col_v, w_v, gbuf, zb, acc_sh, sem):
    c = lax.axis_index("c")
    s = lax.axis_index("s")
    w = c * 16 + s
    ebase = w * ECH

    def zzb(i, _):
        for k in range(3):
            zb[i, pl.ds(k * 16, 16)] = jnp.zeros((16,), jnp.float32)
        return 0
    lax.fori_loop(0, 320, zzb, 0)
    rows0 = s * (NPAD // 16)
    pltpu.sync_copy(zb, acc_sh.at[pl.ds(rows0, 320)])
    pltpu.sync_copy(zb, acc_sh.at[pl.ds(rows0 + 320, 320)])
    plsc.subcore_barrier()

    def batch(b, _):
        base = ebase + b * GB
        d1 = pltpu.async_copy(row_hbm.at[pl.ds(base, GB)], row_v, sem)
        d2 = pltpu.async_copy(col_hbm.at[pl.ds(base, GB)], col_v, sem)
        d3 = pltpu.async_copy(w_hbm.at[pl.ds(base, GB)], w_v.at[pl.ds(0, GB)],
                              sem)
        d1.wait()
        d2.wait()
        d3.wait()
        pltpu.async_copy(h3s_hbm.at[row_v], gbuf, sem).wait()

        def edge(j, _):
            wj = w_v[pl.ds(j, 16)][0]
            for k in range(3):
                gbuf[j, pl.ds(k * 16, 16)] = gbuf[j, pl.ds(k * 16, 16)] * wj
            return 0
        lax.fori_loop(0, GB, edge, 0)
        pltpu.sync_copy(gbuf, acc_sh.at[col_v], add=True)
        return 0

    lax.fori_loop(0, GCB, batch, 0)
    plsc.subcore_barrier()
    pltpu.sync_copy(acc_sh.at[pl.ds(rows0, NPAD // 16)],
                    gpart_hbm.at[c, pl.ds(rows0, NPAD // 16)])


def _gcn_edges(h3s, row, col, ew):
    return pl.kernel(
        _gcn_body,
        out_type=jax.ShapeDtypeStruct((2, NPAD, 48), jnp.float32),
        mesh=_mesh(),
        compiler_params=_SC_PARAMS,
        scratch_types=[
            pltpu.VMEM((GB,), jnp.int32),
            pltpu.VMEM((GB,), jnp.int32),
            pltpu.VMEM((GB + 16,), jnp.float32),
            pltpu.VMEM((GB, 48), jnp.float32),
            pltpu.VMEM((320, 48), jnp.float32),
            pltpu.VMEM_SHARED((NPAD, 48), jnp.float32),
            pltpu.SemaphoreType.DMA,
        ],
    )(h3s, row, col, ew)


def kernel(x, edge_index, edge_weight,
           mlp_kernel0, mlp_bias0, neigh_kernel0, self_kernel0, bias0,
           mlp_kernel1, mlp_bias1, neigh_kernel1, self_kernel1, bias1,
           gcn_kernel, gcn_bias):
    row, col = edge_index[0], edge_index[1]
    xpad = jnp.zeros((NPAD, 128), jnp.float32).at[:N].set(x)

    plist, cnt, degp = _prepass(row, col, edge_weight)

    xm0, fx0 = _l1_transform(xpad, mlp_kernel0, mlp_bias0, self_kernel0)
    part0 = _segmax(xm0, plist, cnt, 64).reshape(NQ, NPAD, 64)

    xm1, fx1 = _l1_finish(fx0, part0, neigh_kernel0, bias0,
                          mlp_kernel1, mlp_bias1, self_kernel1)
    part1 = _segmax(xm1, plist, cnt, 32).reshape(NQ, NPAD, 32)

    gk_pad = jnp.zeros((64, 48), jnp.float32).at[:, :40].set(gcn_kernel)
    h3s, dis = _l2_finish(fx1, part1, degp.reshape(NQ, NPAD), neigh_kernel1,
                          bias1, gk_pad)

    gpart = _gcn_edges(h3s, row, col, edge_weight)

    gb_pad = jnp.zeros((1, 48), jnp.float32).at[0, :40].set(gcn_bias)
    out = _gcn_finish(gpart, h3s, dis, gb_pad)
    return out[:N, :40]


# speedup vs baseline: 14.5832x; 1.0585x over previous
import functools

import jax
import jax.numpy as jnp
from jax import lax
from jax.experimental import pallas as pl
from jax.experimental.pallas import tpu as pltpu
from jax.experimental.pallas import tpu_sc as plsc

N = 10000
NPAD = 10240
NGATH = 16384
E = 320000
NQ = 4
EQ = E // NQ
NH = 8
HN = NPAD // NH
PB = 4000
LCAP = EQ
NCHUNK = 32
ECH = E // NCHUNK
GB = 1000
GCB = ECH // GB

_mesh = functools.partial(
    plsc.VectorSubcoreMesh, core_axis_name="c", subcore_axis_name="s",
    num_cores=2, num_subcores=16)

_SC_PARAMS = pltpu.CompilerParams(
    needs_layout_passes=False, use_tc_tiling_on_sc=False)


def _l1_transform_body(x_ref, mk0_ref, mb0_ref, sk0_ref, xm0_ref, fx0_ref):
    x = x_ref[...]
    xm0_ref[...] = jnp.maximum(
        jnp.dot(x, mk0_ref[...], preferred_element_type=jnp.float32)
        + mb0_ref[...], 0.0)
    fx0_ref[...] = jnp.dot(x, sk0_ref[...], preferred_element_type=jnp.float32)


def _l1_transform(x, mk0, mb0, sk0):
    BN = 1024
    return pl.pallas_call(
        _l1_transform_body,
        grid=(NPAD // BN,),
        in_specs=[
            pl.BlockSpec((BN, 128), lambda i: (i, 0)),
            pl.BlockSpec((128, 64), lambda i: (0, 0)),
            pl.BlockSpec((1, 64), lambda i: (0, 0)),
            pl.BlockSpec((128, 64), lambda i: (0, 0)),
        ],
        out_specs=[
            pl.BlockSpec((BN, 64), lambda i: (i, 0)),
            pl.BlockSpec((BN, 64), lambda i: (i, 0)),
        ],
        out_shape=[
            jax.ShapeDtypeStruct((NGATH, 64), jnp.float32),
            jax.ShapeDtypeStruct((NPAD, 64), jnp.float32),
        ],
    )(x, mk0, mb0.reshape(1, 64), sk0)


def _l1_finish_body(fx0_ref, part_ref, nk0_ref, b0_ref, mk1_ref, mb1_ref,
                    sk1_ref, xm1_ref, fx1_ref):
    red = jnp.max(part_ref[...], axis=0)
    fn = jnp.dot(red, nk0_ref[...], preferred_element_type=jnp.float32)
    h1 = jnp.maximum(
        jnp.concatenate([fx0_ref[...], fn], axis=1) + b0_ref[...], 0.0)
    xm1_ref[...] = jnp.maximum(
        jnp.dot(h1, mk1_ref[...], preferred_element_type=jnp.float32)
        + mb1_ref[...], 0.0)
    fx1_ref[...] = jnp.dot(h1, sk1_ref[...], preferred_element_type=jnp.float32)


def _l1_finish(fx0, part0, nk0, b0, mk1, mb1, sk1):
    BN = 1024
    return pl.pallas_call(
        _l1_finish_body,
        grid=(NPAD // BN,),
        in_specs=[
            pl.BlockSpec((BN, 64), lambda i: (i, 0)),
            pl.BlockSpec((NQ, BN, 64), lambda i: (0, i, 0)),
            pl.BlockSpec((64, 64), lambda i: (0, 0)),
            pl.BlockSpec((1, 128), lambda i: (0, 0)),
            pl.BlockSpec((128, 32), lambda i: (0, 0)),
            pl.BlockSpec((1, 32), lambda i: (0, 0)),
            pl.BlockSpec((128, 32), lambda i: (0, 0)),
        ],
        out_specs=[
            pl.BlockSpec((BN, 32), lambda i: (i, 0)),
            pl.BlockSpec((BN, 32), lambda i: (i, 0)),
        ],
        out_shape=[
            jax.ShapeDtypeStruct((NGATH, 32), jnp.float32),
            jax.ShapeDtypeStruct((NPAD, 32), jnp.float32),
        ],
    )(fx0, part0, nk0, b0.reshape(1, 128), mk1, mb1.reshape(1, 32), sk1)


def _l2_finish_body(fx1_ref, part_ref, degp_ref, nk1_ref, b1_ref, gk_ref,
                    h3s_ref, dis_ref):
    i = pl.program_id(0)
    red = jnp.max(part_ref[...], axis=0)
    fn = jnp.dot(red, nk1_ref[...], preferred_element_type=jnp.float32)
    h2 = jnp.maximum(
        jnp.concatenate([fx1_ref[...], fn], axis=1) + b1_ref[...], 0.0)
    h3 = jnp.dot(h2, gk_ref[...], preferred_element_type=jnp.float32)
    bn = fx1_ref.shape[0]
    rows = i * bn + jax.lax.broadcasted_iota(jnp.int32, (bn, 1), 0)
    deg = jnp.sum(degp_ref[...], axis=0) + jnp.where(rows < N, 1.0, 0.0)
    dis = jnp.where(deg > 0.0, jax.lax.rsqrt(jnp.maximum(deg, 1e-30)), 0.0)
    h3s_ref[...] = h3 * dis
    dis_ref[...] = dis


def _l2_finish(fx1, part1, degp, nk1, b1, gk_pad):
    BN = 1024
    return pl.pallas_call(
        _l2_finish_body,
        grid=(NPAD // BN,),
        in_specs=[
            pl.BlockSpec((BN, 32), lambda i: (i, 0)),
            pl.BlockSpec((NQ, BN, 32), lambda i: (0, i, 0)),
            pl.BlockSpec((NQ, BN, 1), lambda i: (0, i, 0)),
            pl.BlockSpec((32, 32), lambda i: (0, 0)),
            pl.BlockSpec((1, 64), lambda i: (0, 0)),
            pl.BlockSpec((64, 48), lambda i: (0, 0)),
        ],
        out_specs=[
            pl.BlockSpec((BN, 48), lambda i: (i, 0)),
            pl.BlockSpec((BN, 1), lambda i: (i, 0)),
        ],
        out_shape=[
            jax.ShapeDtypeStruct((NPAD, 48), jnp.float32),
            jax.ShapeDtypeStruct((NPAD, 1), jnp.float32),
        ],
    )(fx1, part1, degp.reshape(NQ, NPAD, 1), nk1, b1.reshape(1, 64), gk_pad)


def _gcn_finish_body(gp_ref, h3s_ref, dis_ref, gb_ref, out_ref):
    acc = jnp.sum(gp_ref[...], axis=0)
    out_ref[...] = dis_ref[...] * (acc + h3s_ref[...]) + gb_ref[...]


def _gcn_finish(gpart, h3s, dis, gb_pad):
    BN = 1024
    return pl.pallas_call(
        _gcn_finish_body,
        grid=(NPAD // BN,),
        in_specs=[
            pl.BlockSpec((2, BN, 48), lambda i: (0, i, 0)),
            pl.BlockSpec((BN, 48), lambda i: (i, 0)),
            pl.BlockSpec((BN, 1), lambda i: (i, 0)),
            pl.BlockSpec((1, 48), lambda i: (0, 0)),
        ],
        out_specs=pl.BlockSpec((BN, 48), lambda i: (i, 0)),
        out_shape=jax.ShapeDtypeStruct((NPAD, 48), jnp.float32),
    )(gpart, h3s, dis, gb_pad)


def _prepass_body(row_hbm, col_hbm, w_hbm, plist_hbm, cnt_hbm, degp_hbm,
                  rowb, colb, wb, packed, degloc, cntv, sem):
    w = lax.axis_index("c") * 16 + lax.axis_index("s")
    o = w // NH
    h = w % NH
    lo = h * HN
    ebase = o * EQ

    def zero_deg(i, _):
        degloc[pl.ds(i * 16, 16)] = jnp.zeros((16,), jnp.float32)
        return 0
    lax.fori_loop(0, HN // 16, zero_deg, 0)

    def batch(b, n):
        base = ebase + b * PB
        d1 = pltpu.async_copy(row_hbm.at[pl.ds(base, PB)], rowb, sem)
        d2 = pltpu.async_copy(col_hbm.at[pl.ds(base, PB)], colb, sem)
        d3 = pltpu.async_copy(w_hbm.at[pl.ds(base, PB)], wb, sem)
        d1.wait()
        d2.wait()
        d3.wait()

        def group(g, n):
            r = rowb[pl.ds(g * 16, 16)]
            c = colb[pl.ds(g * 16, 16)]
            wv = wb[pl.ds(g * 16, 16)]
            rl = r - lo
            m = (r >= lo) & (r < lo + HN)
            pk = c | (rl << 14)
            cum = plsc.cumsum(m.astype(jnp.int32))
            plsc.store_scatter(packed, [n + cum - 1], pk, mask=m)
            plsc.addupdate_scatter(degloc, [rl], wv, mask=m)
            return n + plsc.all_reduce_population_count(m)[0]
        return lax.fori_loop(0, PB // 16, group, n)

    n = lax.fori_loop(0, EQ // PB, batch, jnp.int32(0))

    pltpu.sync_copy(packed, plist_hbm.at[pl.ds(w * LCAP, LCAP)])
    cntv[...] = jnp.full((16,), n, jnp.int32)
    pltpu.sync_copy(cntv, cnt_hbm.at[pl.ds(w * 16, 16)])
    pltpu.sync_copy(degloc, degp_hbm.at[pl.ds(o * NPAD + lo, HN)])


def _prepass(row, col, ew):
    return pl.kernel(
        _prepass_body,
        out_type=[
            jax.ShapeDtypeStruct((32 * LCAP + 1024,), jnp.int32),
            jax.ShapeDtypeStruct((32 * 16,), jnp.int32),
            jax.ShapeDtypeStruct((NQ * NPAD,), jnp.float32),
        ],
        mesh=_mesh(),
        compiler_params=_SC_PARAMS,
        scratch_types=[
            pltpu.VMEM((PB,), jnp.int32),
            pltpu.VMEM((PB,), jnp.int32),
            pltpu.VMEM((PB,), jnp.float32),
            pltpu.VMEM((LCAP,), jnp.int32),
            pltpu.VMEM((HN,), jnp.float32),
            pltpu.VMEM((16,), jnp.int32),
            pltpu.SemaphoreType.DMA,
        ],
    )(row, col, ew)


def _segmax_body(nf, plist_hbm, cnt_hbm, part_hbm, cnt_v, pk_v, col_v,
                 gbuf, acc, sem):
    f = nf.shape[1]
    bsz = gbuf.shape[1]
    nb = -(-LCAP // bsz)
    w = lax.axis_index("c") * 16 + lax.axis_index("s")
    o = w // NH
    h = w % NH

    pltpu.sync_copy(cnt_hbm, cnt_v)
    cnt = cnt_v[pl.ds(w * 16, 16)][0]

    def zero(i, _):
        acc[pl.ds(i * 16, 16)] = jnp.zeros((16,), jnp.float32)
        return 0
    lax.fori_loop(0, HN * f // 16, zero, 0)

    def stage(b, q):
        pltpu.sync_copy(plist_hbm.at[pl.ds(w * LCAP + b * bsz, bsz)],
                        pk_v.at[q, pl.ds(0, bsz)])

        def unpk(g, _):
            col_v[q, pl.ds(g * 16, 16)] = pk_v[q, pl.ds(g * 16, 16)] & 16383
            return 0
        lax.fori_loop(0, bsz // 16, unpk, 0)
        pltpu.async_copy(nf.at[col_v.at[q]], gbuf.at[q], sem)

    @pl.when(cnt > 0)
    def _():
        stage(0, 0)

    def batch(b, _):
        p = b % 2

        @pl.when((b + 1) * bsz < cnt)
        def _():
            stage(b + 1, 1 - p)

        @pl.when(b * bsz < cnt)
        def _():
            pltpu.make_async_copy(nf.at[col_v.at[p]], gbuf.at[p], sem).wait()
            m = jnp.minimum(bsz, cnt - b * bsz)

            def group_acc(g, _):
                pkg = pk_v[p, pl.ds(g * 16, 16)]
                for l in range(16):
                    a = (pkg[l] >> 14) * f
                    j = g * 16 + l
                    for k in range(f // 16):
                        acc[pl.ds(a + k * 16, 16)] = jnp.maximum(
                            acc[pl.ds(a + k * 16, 16)],
                            gbuf[p, j, pl.ds(k * 16, 16)])
                return 0
            lax.fori_loop(0, m // 16, group_acc, 0)

            def edge(j, _):
                rl = pk_v[p, pl.ds(j, 16)][0] >> 14
                a = rl * f
                for k in range(f // 16):
                    acc[pl.ds(a + k * 16, 16)] = jnp.maximum(
                        acc[pl.ds(a + k * 16, 16)],
                        gbuf[p, j, pl.ds(k * 16, 16)])
                return 0
            lax.fori_loop((m // 16) * 16, m, edge, 0)
        return 0

    lax.fori_loop(0, nb, batch, 0)
    pltpu.sync_copy(acc, part_hbm.at[pl.ds((o * NPAD + h * HN) * f, HN * f)])


def _segmax(nf, plist, cnt, f):
    bsz = 256 if f == 64 else 768
    return pl.kernel(
        _segmax_body,
        out_type=jax.ShapeDtypeStruct((NQ * NPAD * f,), jnp.float32),
        mesh=_mesh(),
        compiler_params=_SC_PARAMS,
        scratch_types=[
            pltpu.VMEM((32 * 16,), jnp.int32),
            pltpu.VMEM((2, bsz + 16), jnp.int32),
            pltpu.VMEM((2, bsz), jnp.int32),
            pltpu.VMEM((2, bsz, f), jnp.float32),
            pltpu.VMEM((HN * f,), jnp.float32),
            pltpu.SemaphoreType.DMA,
        ],
    )(nf, plist, cnt)


def _gcn_body(h3s_hbm, row_hbm, col_hbm, w_hbm, gpart_hbm,
              row_v, col_v, w_v, gbuf, zb, acc_sh, sem):
    c = lax.axis_index("c")
    s = lax.axis_index("s")
    w = c * 16 + s
    ebase = w * ECH

    def zzb(i, _):
        for k in range(3):
            zb[i, pl.ds(k * 16, 16)] = jnp.zeros((16,), jnp.float32)
        return 0
    lax.fori_loop(0, 320, zzb, 0)
    rows0 = s * (NPAD // 16)
    pltpu.sync_copy(zb, acc_sh.at[pl.ds(rows0, 320)])
    pltpu.sync_copy(zb, acc_sh.at[pl.ds(rows0 + 320, 320)])
    plsc.subcore_barrier()

    def batch(b, _):
        base = ebase + b * GB
        d1 = pltpu.async_copy(row_hbm.at[pl.ds(base, GB)], row_v, sem)
        d2 = pltpu.async_copy(col_hbm.at[pl.ds(base, GB)], col_v, sem)
        d3 = pltpu.async_copy(w_hbm.at[pl.ds(base, GB)], w_v.at[pl.ds(0, GB)],
                              sem)
        d1.wait()
        d2.wait()
        d3.wait()
        pltpu.async_copy(h3s_hbm.at[row_v], gbuf, sem).wait()

        def wgroup(g, _):
            wg = w_v[pl.ds(g * 16, 16)]
            for l in range(16):
                j = g * 16 + l
                for k in range(3):
                    gbuf[j, pl.ds(k * 16, 16)] = (
                        gbuf[j, pl.ds(k * 16, 16)] * wg[l])
            return 0
        lax.fori_loop(0, GB // 16, wgroup, 0)

        def edge(j, _):
            wj = w_v[pl.ds(j, 16)][0]
            for k in range(3):
                gbuf[j, pl.ds(k * 16, 16)] = gbuf[j, pl.ds(k * 16, 16)] * wj
            return 0
        lax.fori_loop((GB // 16) * 16, GB, edge, 0)
        pltpu.sync_copy(gbuf, acc_sh.at[col_v], add=True)
        return 0

    lax.fori_loop(0, GCB, batch, 0)
    plsc.subcore_barrier()
    pltpu.sync_copy(acc_sh.at[pl.ds(rows0, NPAD // 16)],
                    gpart_hbm.at[c, pl.ds(rows0, NPAD // 16)])


def _gcn_edges(h3s, row, col, ew):
    return pl.kernel(
        _gcn_body,
        out_type=jax.ShapeDtypeStruct((2, NPAD, 48), jnp.float32),
        mesh=_mesh(),
        compiler_params=_SC_PARAMS,
        scratch_types=[
            pltpu.VMEM((GB,), jnp.int32),
            pltpu.VMEM((GB,), jnp.int32),
            pltpu.VMEM((GB + 16,), jnp.float32),
            pltpu.VMEM((GB, 48), jnp.float32),
            pltpu.VMEM((320, 48), jnp.float32),
            pltpu.VMEM_SHARED((NPAD, 48), jnp.float32),
            pltpu.SemaphoreType.DMA,
        ],
    )(h3s, row, col, ew)


def kernel(x, edge_index, edge_weight,
           mlp_kernel0, mlp_bias0, neigh_kernel0, self_kernel0, bias0,
           mlp_kernel1, mlp_bias1, neigh_kernel1, self_kernel1, bias1,
           gcn_kernel, gcn_bias):
    row, col = edge_index[0], edge_index[1]
    xpad = jnp.zeros((NPAD, 128), jnp.float32).at[:N].set(x)

    plist, cnt, degp = _prepass(row, col, edge_weight)

    xm0, fx0 = _l1_transform(xpad, mlp_kernel0, mlp_bias0, self_kernel0)
    part0 = _segmax(xm0, plist, cnt, 64).reshape(NQ, NPAD, 64)

    xm1, fx1 = _l1_finish(fx0, part0, neigh_kernel0, bias0,
                          mlp_kernel1, mlp_bias1, self_kernel1)
    part1 = _segmax(xm1, plist, cnt, 32).reshape(NQ, NPAD, 32)

    gk_pad = jnp.zeros((64, 48), jnp.float32).at[:, :40].set(gcn_kernel)
    h3s, dis = _l2_finish(fx1, part1, degp.reshape(NQ, NPAD), neigh_kernel1,
                          bias1, gk_pad)

    gpart = _gcn_edges(h3s, row, col, edge_weight)

    gb_pad = jnp.zeros((1, 48), jnp.float32).at[0, :40].set(gcn_bias)
    out = _gcn_finish(gpart, h3s, dis, gb_pad)
    return out[:N, :40]
